# baseline (speedup 1.0000x reference)
import functools

import jax
import jax.numpy as jnp
from jax import lax
from jax.experimental import pallas as pl
from jax.experimental.pallas import tpu as pltpu
from jax.experimental.pallas import tpu_sc as plsc

_NC = 2
_NS = 16
_NW = _NC * _NS
_CW = 128


@functools.partial(jax.jit, static_argnames=("ch", "dp"))
def _sc_gather(idx, table2, ch, dp):
    mesh = plsc.VectorSubcoreMesh(core_axis_name="c", subcore_axis_name="s")

    @functools.partial(
        pl.kernel,
        mesh=mesh,
        out_type=jax.ShapeDtypeStruct((_NW, ch, _CW, dp), jnp.float32),
        scratch_types=[
            pltpu.VMEM((ch, _CW), jnp.int32),
            pltpu.VMEM((ch, _CW, dp), jnp.float32),
            pltpu.SemaphoreType.DMA,
        ],
    )
    def k(idx_hbm, table_hbm, out_hbm, idx_v, rows_v, sem):
        wid = lax.axis_index("s") * _NC + lax.axis_index("c")
        pltpu.sync_copy(idx_hbm.at[wid], idx_v)
        copies = [
            pltpu.async_copy(table_hbm.at[idx_v.at[j]], rows_v.at[j], sem)
            for j in range(ch)
        ]
        for c in copies:
            c.wait()
        pltpu.sync_copy(rows_v, out_hbm.at[wid])

    return k(idx, table2)


def kernel(batch, table):
    (b,) = batch.shape
    v, d = table.shape
    table2 = table.reshape(v // 2, 2 * d)
    idx = batch.astype(jnp.int32)
    ch = b // (_NW * _CW)
    i2 = (idx >> 1).reshape(_NW, ch, _CW)
    pairs = _sc_gather(i2, table2, ch, 2 * d).reshape(b, 2 * d)
    hi = (idx & 1)[:, None] == 1
    out = jnp.where(hi, pairs[:, d:], pairs[:, :d])
    return out.reshape(b, 1, d)


# speedup vs baseline: 1.7342x; 1.7342x over previous
import functools

import jax
import jax.numpy as jnp
from jax import lax
from jax.experimental import pallas as pl
from jax.experimental.pallas import tpu as pltpu
from jax.experimental.pallas import tpu_sc as plsc

_NC = 2
_NS = 16
_NW = _NC * _NS
_CW = 128
_TB = 2048


def _tpose_body(x1_ref, x2_ref, o_ref):
    d = x1_ref.shape[0]
    o_ref[:, :d] = x1_ref[...].T
    o_ref[:, d:] = x2_ref[...].T


@jax.jit
def _tc_pair_table(tt):
    d, v = tt.shape
    g = pl.cdiv(v, 2 * _TB)
    nb = pl.cdiv(v, _TB)
    return pl.pallas_call(
        _tpose_body,
        grid=(g,),
        in_specs=[
            pl.BlockSpec((d, _TB), lambda k: (0, 2 * k)),
            pl.BlockSpec(
                (d, _TB), lambda k, nb=nb: (0, jnp.minimum(2 * k + 1, nb - 1))
            ),
        ],
        out_specs=pl.BlockSpec((_TB, 2 * d), lambda k: (k, 0)),
        out_shape=jax.ShapeDtypeStruct((g * _TB, 2 * d), jnp.float32),
    )(tt, tt)


@functools.partial(jax.jit, static_argnames=("ch", "dp"))
def _sc_gather(idx, table2, ch, dp):
    mesh = plsc.VectorSubcoreMesh(core_axis_name="c", subcore_axis_name="s")

    @functools.partial(
        pl.kernel,
        mesh=mesh,
        out_type=jax.ShapeDtypeStruct((_NW, ch, _CW, dp), jnp.float32),
        scratch_types=[
            pltpu.VMEM((ch, _CW), jnp.int32),
            pltpu.VMEM((ch, _CW, dp), jnp.float32),
            pltpu.SemaphoreType.DMA,
        ],
    )
    def k(idx_hbm, table_hbm, out_hbm, idx_v, rows_v, sem):
        wid = lax.axis_index("s") * _NC + lax.axis_index("c")
        pltpu.sync_copy(idx_hbm.at[wid], idx_v)
        copies = [
            pltpu.async_copy(table_hbm.at[idx_v.at[j]], rows_v.at[j], sem)
            for j in range(ch)
        ]
        for c in copies:
            c.wait()
        pltpu.sync_copy(rows_v, out_hbm.at[wid])

    return k(idx, table2)


def kernel(batch, table):
    (b,) = batch.shape
    v, d = table.shape
    table2 = _tc_pair_table(table.T)
    idx = batch.astype(jnp.int32)
    ch = b // (_NW * _CW)
    i2 = ((idx // (2 * _TB)) * _TB + (idx % _TB)).reshape(_NW, ch, _CW)
    pairs = _sc_gather(i2, table2, ch, 2 * d).reshape(b, 2 * d)
    hi = ((idx // _TB) & 1)[:, None] == 1
    out = jnp.where(hi, pairs[:, d:], pairs[:, :d])
    return out.reshape(b, 1, d)
